# Initial kernel scaffold; baseline (speedup 1.0000x reference)
import jax
import jax.numpy as jnp
from jax.experimental import pallas as pl


def kernel(x, W1, b1, W2, b2, codebook, W3, b3, W4, b4):
    raise NotImplementedError("write your pallas kernel here")


# speedup vs baseline: 3.4841x; 3.4841x over previous
import functools
import math

import jax
import jax.numpy as jnp
from jax import lax
from jax.experimental import pallas as pl
from jax.experimental.pallas import tpu as pltpu
from jax.experimental.pallas import tpu_sc as plsc

_SQRT_HALF = math.sqrt(0.5)


def _gelu_exact(v):
    return 0.5 * v * (1.0 + lax.erf(v * _SQRT_HALF))


def _dot(a, b, dims=None):
    if dims is None:
        return jnp.dot(a, b, preferred_element_type=jnp.float32)
    return lax.dot_general(a, b, (dims, ((), ())),
                           preferred_element_type=jnp.float32)


def _cb_body(cb_ref, w3_ref, b3_ref, w4_ref, b4_ref, cn_ref, g_ref):
    cb = cb_ref[...]
    n = jnp.sqrt(jnp.sum(cb * cb, axis=1, keepdims=True))
    cn = cb / jnp.clip(n, 1e-12, None)
    cn_ref[...] = cn
    hh = _dot(cn, w3_ref[...]) + b3_ref[...]
    hh = _gelu_exact(hh)
    g_ref[...] = _dot(hh, w4_ref[...]) + b4_ref[...]


def _codebook_prep(codebook, W3, b3, W4, b4, kb):
    K, D = codebook.shape
    H = W3.shape[1]
    Dout = W4.shape[1]
    return pl.pallas_call(
        _cb_body,
        grid=(K // kb,),
        in_specs=[
            pl.BlockSpec((kb, D), lambda t: (t, 0)),
            pl.BlockSpec((D, H), lambda t: (0, 0)),
            pl.BlockSpec((1, H), lambda t: (0, 0)),
            pl.BlockSpec((H, Dout), lambda t: (0, 0)),
            pl.BlockSpec((1, Dout), lambda t: (0, 0)),
        ],
        out_specs=[
            pl.BlockSpec((kb, D), lambda t: (t, 0)),
            pl.BlockSpec((kb, Dout), lambda t: (t, 0)),
        ],
        out_shape=[
            jax.ShapeDtypeStruct((K, D), jnp.float32),
            jax.ShapeDtypeStruct((K, Dout), jnp.float32),
        ],
        compiler_params=pltpu.CompilerParams(
            dimension_semantics=("arbitrary",)),
    )(codebook, W3, b3.reshape(1, H), W4, b4.reshape(1, Dout))


def _enc_body(x_ref, w1_ref, b1_ref, w2_ref, b2_ref, cn_ref,
              ind_ref, acc_ref, *, kb, n_kb):
    t = pl.program_id(0)
    h1 = _dot(x_ref[...], w1_ref[...]) + b1_ref[...]
    h1 = _gelu_exact(h1)
    h = _dot(h1, w2_ref[...]) + b2_ref[...]
    T = h.shape[0]

    hc = jnp.clip(jnp.sqrt(jnp.sum(h * h, axis=1, keepdims=True)),
                  1e-12, None)
    xn = h / hc

    mx = jnp.full((1, T), -jnp.inf, dtype=jnp.float32)
    ind = jnp.zeros((1, T), dtype=jnp.int32)
    nsq = jnp.zeros((1, T), dtype=jnp.float32)
    for k in range(n_kb):
        cnb = cn_ref[pl.ds(k * kb, kb), :]
        s = _dot(cnb, xn, dims=((1,), (1,)))
        bm = jnp.max(s, axis=0, keepdims=True)
        iota = lax.broadcasted_iota(jnp.int32, (kb, T), 0)
        bi = jnp.min(jnp.where(s == bm, iota, kb),
                     axis=0, keepdims=True)
        nsqb = jnp.sum(cnb * cnb, axis=1, keepdims=True)
        bn = jnp.max(jnp.where(iota == bi, nsqb, -jnp.inf),
                     axis=0, keepdims=True)
        upd = bm > mx
        mx = jnp.where(upd, bm, mx)
        ind = jnp.where(upd, bi + k * kb, ind)
        nsq = jnp.where(upd, bn, nsq)

    ind_ref[...] = ind.reshape(1, 1, T)
    qh = _dot(mx, hc)[0, 0]
    part = jnp.sum(nsq) - 2.0 * qh + jnp.sum(h * h)

    @pl.when(t == 0)
    def _init():
        acc_ref[0, 0] = 0.0

    acc_ref[0, 0] += part


def _encode_argmax(xf, W1, b1, W2, b2, cn, tb, kb):
    N, Din = xf.shape
    H = W1.shape[1]
    D = W2.shape[1]
    K = cn.shape[0]
    nt = N // tb
    body = functools.partial(_enc_body, kb=kb, n_kb=K // kb)
    return pl.pallas_call(
        body,
        grid=(nt,),
        in_specs=[
            pl.BlockSpec((tb, Din), lambda t: (t, 0)),
            pl.BlockSpec((Din, H), lambda t: (0, 0)),
            pl.BlockSpec((1, H), lambda t: (0, 0)),
            pl.BlockSpec((H, D), lambda t: (0, 0)),
            pl.BlockSpec((1, D), lambda t: (0, 0)),
            pl.BlockSpec((K, D), lambda t: (0, 0)),
        ],
        out_specs=[
            pl.BlockSpec((1, 1, tb), lambda t: (t, 0, 0)),
            pl.BlockSpec((1, 1), lambda t: (0, 0),
                         memory_space=pltpu.SMEM),
        ],
        out_shape=[
            jax.ShapeDtypeStruct((nt, 1, tb), jnp.int32),
            jax.ShapeDtypeStruct((1, 1), jnp.float32),
        ],
        compiler_params=pltpu.CompilerParams(
            dimension_semantics=("arbitrary",)),
    )(xf, W1, b1.reshape(1, H), W2, b2.reshape(1, D), cn)


def _sc_gather(G, ind, n_chunks):
    K, Dout = G.shape
    (N,) = ind.shape
    info = plsc.get_sparse_core_info()
    NC, NS = info.num_cores, info.num_subcores
    NW = NC * NS
    b_per_w = N // NW
    ch = b_per_w // n_chunks
    mesh = plsc.VectorSubcoreMesh(core_axis_name="c", subcore_axis_name="s")

    @functools.partial(
        pl.kernel, mesh=mesh,
        out_type=jax.ShapeDtypeStruct((N, Dout), jnp.float32),
        scratch_types=[
            pltpu.VMEM((ch,), jnp.int32),
            pltpu.VMEM((ch, Dout), jnp.float32),
            pltpu.SemaphoreType.DMA,
        ],
    )
    def k(g_hbm, idx_hbm, out_hbm, idx_v, rows_v, sem):
        wid = lax.axis_index("s") * NC + lax.axis_index("c")
        base = wid * b_per_w
        for c in range(n_chunks):
            off = base + c * ch
            pltpu.sync_copy(idx_hbm.at[pl.ds(off, ch)], idx_v)
            pltpu.async_copy(g_hbm.at[idx_v], rows_v, sem).wait()
            pltpu.sync_copy(rows_v, out_hbm.at[pl.ds(off, ch)])

    return k(G, ind)


def kernel(x, W1, b1, W2, b2, codebook, W3, b3, W4, b4):
    B, S, Din = x.shape
    N = B * S
    K, D = codebook.shape
    Dout = W4.shape[1]

    xf = x.reshape(N, Din)
    cn, G = _codebook_prep(codebook, W3, b3, W4, b4, kb=1024)
    ind3, acc = _encode_argmax(xf, W1, b1, W2, b2, cn, tb=512, kb=512)
    ind = ind3.reshape(N)
    o = _sc_gather(G, ind, n_chunks=4)
    loss = acc[0, 0] / jnp.float32(N * D)
    return o.reshape(B, S, Dout), loss
